# Initial kernel scaffold; baseline (speedup 1.0000x reference)
import jax
import jax.numpy as jnp
from jax.experimental import pallas as pl


def kernel(tokens, W, b, codebook):
    raise NotImplementedError("write your pallas kernel here")


# speedup vs baseline: 1.0006x; 1.0006x over previous
import jax
import jax.numpy as jnp
from jax.experimental import pallas as pl
from jax.experimental.pallas import tpu as pltpu

_B = 8192
_TD = 768
_HD = 256
_K = 8192
_BM = 256


def _vq_kernel(tokens_ref, wt_ref, b_ref, cb_ref, ones_ref, out_ref):
    h = (
        jax.lax.dot_general(
            tokens_ref[...],
            wt_ref[...],
            (((1,), (0,)), ((), ())),
            preferred_element_type=jnp.float32,
        )
        + b_ref[...]
    )
    h2 = jax.lax.dot_general(
        ones_ref[...],
        h * h,
        (((1,), (1,)), ((), ())),
        preferred_element_type=jnp.float32,
    )
    cbv = cb_ref[...]
    c2 = jnp.sum(cbv * cbv, axis=1, keepdims=True)
    hm = (-2.0 * h).astype(jnp.bfloat16)
    st = jax.lax.dot_general(
        cbv,
        hm,
        (((1,), (1,)), ((), ())),
        preferred_element_type=jnp.float32,
    )
    d2 = jnp.maximum((h2 + c2) + st, 0.0)
    dist = jnp.sqrt(d2)
    minv = jnp.min(dist, axis=0, keepdims=True)
    idx = jax.lax.broadcasted_iota(jnp.int32, (_K, _BM), 0)
    win = jnp.where(dist == minv, idx, _K)
    out_ref[...] = jnp.min(win, axis=0, keepdims=True)


def kernel(tokens, W, b, codebook):
    wt = W.T
    b2 = b.reshape(1, _HD)
    ones = jnp.ones((1, _HD), jnp.float32)
    out = pl.pallas_call(
        _vq_kernel,
        grid=(_B // _BM,),
        in_specs=[
            pl.BlockSpec((_BM, _TD), lambda i: (i, 0)),
            pl.BlockSpec((_TD, _HD), lambda i: (0, 0)),
            pl.BlockSpec((1, _HD), lambda i: (0, 0)),
            pl.BlockSpec((_K, _HD), lambda i: (0, 0)),
            pl.BlockSpec((1, _HD), lambda i: (0, 0)),
        ],
        out_specs=pl.BlockSpec((1, _BM), lambda i: (0, i)),
        out_shape=jax.ShapeDtypeStruct((1, _B), jnp.int32),
    )(tokens, wt, b2, codebook, ones)
    return out.reshape(_B)
